# Initial kernel scaffold; baseline (speedup 1.0000x reference)
import jax
import jax.numpy as jnp
from jax.experimental import pallas as pl


def kernel(x, G0, G1, G2):
    raise NotImplementedError("write your pallas kernel here")


# speedup vs baseline: 3.3108x; 3.3108x over previous
import functools

import jax
import jax.numpy as jnp
import numpy as np
from jax import lax
from jax.experimental import pallas as pl
from jax.experimental.pallas import tpu as pltpu
from jax.experimental.pallas import tpu_sc as plsc

VOCAB_D = (100, 100, 100)
A, RDIM, DDIM = 8, 32, 16


def _mm_body(lhs_ref, rhs_ref, out_ref):
    out_ref[...] = jnp.dot(
        lhs_ref[...], rhs_ref[...], preferred_element_type=jnp.float32
    )


def _pair_table(G1, G2):
    lhs = jnp.transpose(G1, (1, 0, 2, 3)).reshape(12800, 32)
    rhs = G2.reshape(32, 400)
    out = pl.pallas_call(
        _mm_body,
        grid=(10,),
        in_specs=[
            pl.BlockSpec((1280, 32), lambda i: (i, 0)),
            pl.BlockSpec((32, 400), lambda i: (0, 0)),
        ],
        out_specs=pl.BlockSpec((1280, 400), lambda i: (i, 0)),
        out_shape=jax.ShapeDtypeStruct((12800, 400), jnp.float32),
    )(lhs, rhs)
    return (
        out.reshape(100, 32, 4, 100, 4)
        .transpose(0, 3, 1, 2, 4)
        .reshape(10000, 512)
    )


T_TOKENS = 4096 * 26
NWORKERS = 32
TOK_PER_W = T_TOKENS // NWORKERS
CHUNK = 32
NCHUNKS = TOK_PER_W // CHUNK


def _sc_body(x_hbm, b12_hbm, t0_hbm, out_hbm,
             t0_v, idx_v, i0_v, i12_v, rows_v, out_v, gsem):
    nc = 2
    wid = lax.axis_index("s") * nc + lax.axis_index("c")
    base = wid * TOK_PER_W

    pltpu.sync_copy(t0_hbm, t0_v)
    iota = lax.iota(jnp.int32, 16)

    def chunk_body(ci, _):
        tok0 = base + ci * CHUNK
        pltpu.sync_copy(x_hbm.at[pl.ds(tok0, CHUNK)], idx_v)

        for v in range(CHUNK // 16):
            ids = idx_v[pl.ds(v * 16, 16)]
            q = (ids.astype(jnp.float32) * np.float32(1e-4)).astype(jnp.int32)
            r = ids - q * 10000
            q = jnp.where(r >= 10000, q + 1, q)
            q = jnp.where(r < 0, q - 1, q)
            r = ids - q * 10000
            i0_v[pl.ds(v * 16, 16)] = q
            i12_v[pl.ds(v * 16, 16)] = r

        pltpu.async_copy(b12_hbm.at[i12_v], rows_v, gsem).wait()

        def tok_body(t, _):
            i0s = plsc.load_gather(i0_v, [jnp.full((16,), t, jnp.int32)])
            sidx0 = i0s * 256
            bvecs = [rows_v[t, pl.ds(r * 16, 16)] for r in range(RDIM)]
            for a in range(A):
                sa0 = plsc.load_gather(t0_v, [sidx0 + (iota + a * 32)])
                sa1 = plsc.load_gather(t0_v, [sidx0 + (iota + (a * 32 + 16))])
                acc = bvecs[0] * sa0[0]
                for r in range(1, RDIM):
                    s = sa0[r] if r < 16 else sa1[r - 16]
                    acc = acc + bvecs[r] * s
                out_v[t, pl.ds(a * 16, 16)] = acc
            return ()

        lax.fori_loop(0, CHUNK, tok_body, (), unroll=False)
        pltpu.sync_copy(out_v, out_hbm.at[pl.ds(tok0, CHUNK)])
        return ()

    lax.fori_loop(0, NCHUNKS, chunk_body, (), unroll=False)


def _sc_lookup(x_flat, b12, t0_flat):
    mesh = plsc.VectorSubcoreMesh(core_axis_name="c", subcore_axis_name="s")
    kern = pl.kernel(
        _sc_body,
        out_type=jax.ShapeDtypeStruct((T_TOKENS, 128), jnp.float32),
        mesh=mesh,
        scratch_types=[
            pltpu.VMEM((25600,), jnp.float32),
            pltpu.VMEM((CHUNK,), jnp.int32),
            pltpu.VMEM((CHUNK,), jnp.int32),
            pltpu.VMEM((CHUNK,), jnp.int32),
            pltpu.VMEM((CHUNK, 512), jnp.float32),
            pltpu.VMEM((CHUNK, 128), jnp.float32),
            pltpu.SemaphoreType.DMA,
        ],
        compiler_params=pltpu.CompilerParams(needs_layout_passes=False),
    )
    return kern(x_flat, b12, t0_flat)


def kernel(x, G0, G1, G2):
    b12 = _pair_table(G1, G2)
    t0_flat = G0.reshape(25600)
    x_flat = x.reshape(T_TOKENS).astype(jnp.int32)
    out = _sc_lookup(x_flat, b12, t0_flat)
    return out.reshape(x.shape + (128,))


# speedup vs baseline: 4.3178x; 1.3042x over previous
import functools

import jax
import jax.numpy as jnp
import numpy as np
from jax import lax
from jax.experimental import pallas as pl
from jax.experimental.pallas import tpu as pltpu
from jax.experimental.pallas import tpu_sc as plsc

VOCAB_D = (100, 100, 100)
A, RDIM, DDIM = 8, 32, 16


def _mm_body(lhs_ref, rhs_ref, out_ref):
    out_ref[...] = jnp.dot(
        lhs_ref[...], rhs_ref[...], preferred_element_type=jnp.float32
    )


def _pair_table(G1, G2):
    lhs = jnp.transpose(G1, (1, 0, 2, 3)).reshape(12800, 32)
    rhs = G2.reshape(32, 400)
    out = pl.pallas_call(
        _mm_body,
        grid=(10,),
        in_specs=[
            pl.BlockSpec((1280, 32), lambda i: (i, 0)),
            pl.BlockSpec((32, 400), lambda i: (0, 0)),
        ],
        out_specs=pl.BlockSpec((1280, 400), lambda i: (i, 0)),
        out_shape=jax.ShapeDtypeStruct((12800, 400), jnp.float32),
    )(lhs, rhs)
    return (
        out.reshape(100, 32, 4, 100, 4)
        .transpose(0, 3, 1, 2, 4)
        .reshape(10000, 512)
    )


T_TOKENS = 4096 * 26
NWORKERS = 32
TOK_PER_W = T_TOKENS // NWORKERS
CHUNK = 32
NCHUNKS = TOK_PER_W // CHUNK


def _sc_body(x_hbm, b12_hbm, t0_hbm, out_hbm,
             t0_v, idx_v, i0_v, i12_v, rows_v, out_v, gsem):
    nc = 2
    wid = lax.axis_index("s") * nc + lax.axis_index("c")
    base = wid * TOK_PER_W

    pltpu.sync_copy(t0_hbm, t0_v)
    iota = lax.iota(jnp.int32, 16)

    def stage_and_start(ci, slot):
        tok0 = base + ci * CHUNK
        pltpu.sync_copy(x_hbm.at[pl.ds(tok0, CHUNK)], idx_v)
        for v in range(CHUNK // 16):
            ids = idx_v[pl.ds(v * 16, 16)]
            q = (ids.astype(jnp.float32) * np.float32(1e-4)).astype(jnp.int32)
            r = ids - q * 10000
            q = jnp.where(r >= 10000, q + 1, q)
            q = jnp.where(r < 0, q - 1, q)
            r = ids - q * 10000
            i0_v[pl.ds(slot * CHUNK + v * 16, 16)] = q
            i12_v[slot, pl.ds(v * 16, 16)] = r
        pltpu.async_copy(
            b12_hbm.at[i12_v.at[slot]], rows_v.at[slot], gsem.at[slot]
        )

    def compute(ci, slot):
        tok0 = base + ci * CHUNK
        pltpu.make_async_copy(
            b12_hbm.at[i12_v.at[slot]], rows_v.at[slot], gsem.at[slot]
        ).wait()

        def tok_body(t, _):
            i0s = plsc.load_gather(
                i0_v, [jnp.full((16,), slot * CHUNK + t, jnp.int32)]
            )
            sidx0 = i0s * 256
            srow = [
                plsc.load_gather(t0_v, [sidx0 + (iota + 16 * v)])
                for v in range(16)
            ]
            acc = [[None, None] for _ in range(A)]
            for r in range(RDIM):
                bvec = rows_v[slot, t, pl.ds(r * 16, 16)]
                h = r & 1
                for a in range(A):
                    p = bvec * srow[2 * a + (r // 16)][r % 16]
                    acc[a][h] = p if acc[a][h] is None else acc[a][h] + p
            for a in range(A):
                out_v[t, pl.ds(a * 16, 16)] = acc[a][0] + acc[a][1]
            return ()

        lax.fori_loop(0, CHUNK, tok_body, (), unroll=False)
        pltpu.sync_copy(out_v, out_hbm.at[pl.ds(tok0, CHUNK)])

    stage_and_start(0, 0)

    def pair_body(k, _):
        ci0 = 2 * k
        stage_and_start(ci0 + 1, 1)
        compute(ci0, 0)

        @pl.when(k < NCHUNKS // 2 - 1)
        def _():
            stage_and_start(ci0 + 2, 0)

        compute(ci0 + 1, 1)
        return ()

    lax.fori_loop(0, NCHUNKS // 2, pair_body, (), unroll=False)


def _sc_lookup(x_flat, b12, t0_flat):
    mesh = plsc.VectorSubcoreMesh(core_axis_name="c", subcore_axis_name="s")
    kern = pl.kernel(
        _sc_body,
        out_type=jax.ShapeDtypeStruct((T_TOKENS, 128), jnp.float32),
        mesh=mesh,
        scratch_types=[
            pltpu.VMEM((25600,), jnp.float32),
            pltpu.VMEM((CHUNK,), jnp.int32),
            pltpu.VMEM((2 * CHUNK,), jnp.int32),
            pltpu.VMEM((2, CHUNK), jnp.int32),
            pltpu.VMEM((2, CHUNK, 512), jnp.float32),
            pltpu.VMEM((CHUNK, 128), jnp.float32),
            pltpu.SemaphoreType.DMA((2,)),
        ],
        compiler_params=pltpu.CompilerParams(needs_layout_passes=False),
    )
    return kern(x_flat, b12, t0_flat)


def kernel(x, G0, G1, G2):
    b12 = _pair_table(G1, G2)
    t0_flat = G0.reshape(25600)
    x_flat = x.reshape(T_TOKENS).astype(jnp.int32)
    out = _sc_lookup(x_flat, b12, t0_flat)
    return out.reshape(x.shape + (128,))


# speedup vs baseline: 4.3694x; 1.0119x over previous
import functools

import jax
import jax.numpy as jnp
import numpy as np
from jax import lax
from jax.experimental import pallas as pl
from jax.experimental.pallas import tpu as pltpu
from jax.experimental.pallas import tpu_sc as plsc

VOCAB_D = (100, 100, 100)
A, RDIM, DDIM = 8, 32, 16


def _mm_body(lhs_ref, rhs_ref, out_ref):
    out_ref[...] = jnp.dot(
        lhs_ref[...], rhs_ref[...], preferred_element_type=jnp.float32
    )


def _pair_table(G1, G2):
    lhs = jnp.transpose(G1, (1, 0, 2, 3)).reshape(12800, 32)
    rhs = G2.reshape(32, 400)
    out = pl.pallas_call(
        _mm_body,
        grid=(10,),
        in_specs=[
            pl.BlockSpec((1280, 32), lambda i: (i, 0)),
            pl.BlockSpec((32, 400), lambda i: (0, 0)),
        ],
        out_specs=pl.BlockSpec((1280, 400), lambda i: (i, 0)),
        out_shape=jax.ShapeDtypeStruct((12800, 400), jnp.float32),
    )(lhs, rhs)
    return (
        out.reshape(100, 32, 4, 100, 4)
        .transpose(0, 3, 1, 2, 4)
        .reshape(10000, 512)
    )


T_TOKENS = 4096 * 26
NWORKERS = 32
TOK_PER_W = T_TOKENS // NWORKERS
CHUNK = 64
NCHUNKS = TOK_PER_W // CHUNK


def _sc_body(x_hbm, b12_hbm, t0_hbm, out_hbm,
             t0_v, idx_v, i0_v, i12_v, rows_v, out_v, gsem):
    nc = 2
    wid = lax.axis_index("s") * nc + lax.axis_index("c")
    base = wid * TOK_PER_W

    pltpu.sync_copy(t0_hbm, t0_v)
    iota = lax.iota(jnp.int32, 16)

    def stage_and_start(ci, slot):
        tok0 = base + ci * CHUNK
        pltpu.sync_copy(x_hbm.at[pl.ds(tok0, CHUNK)], idx_v)
        for v in range(CHUNK // 16):
            ids = idx_v[pl.ds(v * 16, 16)]
            q = (ids.astype(jnp.float32) * np.float32(1e-4)).astype(jnp.int32)
            r = ids - q * 10000
            q = jnp.where(r >= 10000, q + 1, q)
            q = jnp.where(r < 0, q - 1, q)
            r = ids - q * 10000
            i0_v[pl.ds(slot * CHUNK + v * 16, 16)] = q
            i12_v[slot, pl.ds(v * 16, 16)] = r
        pltpu.async_copy(
            b12_hbm.at[i12_v.at[slot]], rows_v.at[slot], gsem.at[slot]
        )

    def compute(ci, slot):
        tok0 = base + ci * CHUNK
        pltpu.make_async_copy(
            b12_hbm.at[i12_v.at[slot]], rows_v.at[slot], gsem.at[slot]
        ).wait()

        def tok_body(t, _):
            i0s = plsc.load_gather(
                i0_v, [jnp.full((16,), slot * CHUNK + t, jnp.int32)]
            )
            sidx0 = i0s * 256
            srow = [
                plsc.load_gather(t0_v, [sidx0 + (iota + 16 * v)])
                for v in range(16)
            ]
            acc = [[None, None] for _ in range(A)]
            for r in range(RDIM):
                bvec = rows_v[slot, t, pl.ds(r * 16, 16)]
                h = r & 1
                for a in range(A):
                    p = bvec * srow[2 * a + (r // 16)][r % 16]
                    acc[a][h] = p if acc[a][h] is None else acc[a][h] + p
            for a in range(A):
                out_v[t, pl.ds(a * 16, 16)] = acc[a][0] + acc[a][1]
            return ()

        lax.fori_loop(0, CHUNK, tok_body, (), unroll=2)
        pltpu.sync_copy(out_v, out_hbm.at[pl.ds(tok0, CHUNK)])

    stage_and_start(0, 0)

    def pair_body(k, _):
        ci0 = 2 * k
        stage_and_start(ci0 + 1, 1)
        compute(ci0, 0)

        @pl.when(k < NCHUNKS // 2 - 1)
        def _():
            stage_and_start(ci0 + 2, 0)

        compute(ci0 + 1, 1)
        return ()

    lax.fori_loop(0, NCHUNKS // 2, pair_body, (), unroll=False)


def _sc_lookup(x_flat, b12, t0_flat):
    mesh = plsc.VectorSubcoreMesh(core_axis_name="c", subcore_axis_name="s")
    kern = pl.kernel(
        _sc_body,
        out_type=jax.ShapeDtypeStruct((T_TOKENS, 128), jnp.float32),
        mesh=mesh,
        scratch_types=[
            pltpu.VMEM((25600,), jnp.float32),
            pltpu.VMEM((CHUNK,), jnp.int32),
            pltpu.VMEM((2 * CHUNK,), jnp.int32),
            pltpu.VMEM((2, CHUNK), jnp.int32),
            pltpu.VMEM((2, CHUNK, 512), jnp.float32),
            pltpu.VMEM((CHUNK, 128), jnp.float32),
            pltpu.SemaphoreType.DMA((2,)),
        ],
        compiler_params=pltpu.CompilerParams(needs_layout_passes=False),
    )
    return kern(x_flat, b12, t0_flat)


def kernel(x, G0, G1, G2):
    b12 = _pair_table(G1, G2)
    t0_flat = G0.reshape(25600)
    x_flat = x.reshape(T_TOKENS).astype(jnp.int32)
    out = _sc_lookup(x_flat, b12, t0_flat)
    return out.reshape(x.shape + (128,))
